# baseline (speedup 1.0000x reference)
import jax
import jax.numpy as jnp
from jax.experimental import pallas as pl
from jax.experimental.pallas import tpu as pltpu

_N = 2048
_F = 256
_LEV = 2
_R = 2
_NOP = _LEV * _R - 1
_NS = _NOP * _N

_BM1 = 512
_BM2 = 256
_BK2 = 512


def _z_kernel(c_ref, rv_ref, d_ref, x_ref, z_ref):
    z_ref[...] = (c_ref[0] * rv_ref[...]) * jnp.dot(
        d_ref[...], x_ref[...], preferred_element_type=jnp.float32)


def _out_kernel(c_ref, adj_ref, dl_ref, z_ref, x_ref, h0_ref, w_ref, o_ref):
    k = pl.program_id(1)

    @pl.when(k == 0)
    def _():
        o_ref[...] = jnp.zeros_like(o_ref)

    part = c_ref[1] * jnp.dot(adj_ref[...], x_ref[...],
                              preferred_element_type=jnp.float32)
    for i in range(_NOP):
        part += jnp.dot(dl_ref[i], z_ref[i], preferred_element_type=jnp.float32)
    o_ref[...] += part

    @pl.when(k == pl.num_programs(1) - 1)
    def _():
        s = c_ref[3] * o_ref[...] + c_ref[2] * h0_ref[...]
        o_ref[...] = (c_ref[4] * jnp.dot(s, w_ref[...],
                                         preferred_element_type=jnp.float32)
                      + c_ref[5] * s)


def kernel(input, adj, d_list, h0, weight, lamda, alpha, l, gamma):
    x = input
    dl = d_list[1:]
    d2 = dl.reshape(_NS, _N)
    rv2 = jax.random.uniform(jax.random.key(42), (_LEV * _R * _N, 1),
                             dtype=jnp.float32)[_N:]
    theta = jnp.log(lamda / l + 1)
    g = jnp.asarray(gamma, jnp.float32)
    a = jnp.asarray(alpha, jnp.float32)
    t = jnp.asarray(theta, jnp.float32)
    c = jnp.stack([g, 1 - g, a, 1 - a, t, 1 - t]).astype(jnp.float32)

    z = pl.pallas_call(
        _z_kernel,
        grid=(_NS // _BM1,),
        in_specs=[
            pl.BlockSpec(memory_space=pltpu.SMEM),
            pl.BlockSpec((_BM1, 1), lambda m: (m, 0)),
            pl.BlockSpec((_BM1, _N), lambda m: (m, 0)),
            pl.BlockSpec((_N, _F), lambda m: (0, 0)),
        ],
        out_specs=pl.BlockSpec((_BM1, _F), lambda m: (m, 0)),
        out_shape=jax.ShapeDtypeStruct((_NS, _F), jnp.float32),
    )(c, rv2, d2, x)

    z3 = z.reshape(_NOP, _N, _F)

    out = pl.pallas_call(
        _out_kernel,
        grid=(_N // _BM2, _N // _BK2),
        in_specs=[
            pl.BlockSpec(memory_space=pltpu.SMEM),
            pl.BlockSpec((_BM2, _BK2), lambda m, k: (m, k)),
            pl.BlockSpec((_NOP, _BM2, _BK2), lambda m, k: (0, m, k)),
            pl.BlockSpec((_NOP, _BK2, _F), lambda m, k: (0, k, 0)),
            pl.BlockSpec((_BK2, _F), lambda m, k: (k, 0)),
            pl.BlockSpec((_BM2, _F), lambda m, k: (m, 0)),
            pl.BlockSpec((_F, _F), lambda m, k: (0, 0)),
        ],
        out_specs=pl.BlockSpec((_BM2, _F), lambda m, k: (m, 0)),
        out_shape=jax.ShapeDtypeStruct((_N, _F), jnp.float32),
    )(c, adj, dl, z3, x, h0, weight)
    return out


# speedup vs baseline: 1.3304x; 1.3304x over previous
import jax
import jax.numpy as jnp
from jax.experimental import pallas as pl
from jax.experimental.pallas import tpu as pltpu

_N = 2048
_F = 256
_LEV = 2
_R = 2
_NOP = _LEV * _R - 1
_NS = _NOP * _N

_BM1 = 512
_BM2 = 256
_BK2 = 512


def _z_kernel(c_ref, rv_ref, d_ref, x_ref, z_ref):
    z_ref[...] = (c_ref[0] * rv_ref[...]) * jnp.dot(
        d_ref[0], x_ref[...], preferred_element_type=jnp.float32)


def _out_kernel(c_ref, adj_ref, d1_ref, d2_ref, d3_ref, z_ref, x_ref, h0_ref,
                w_ref, o_ref):
    k = pl.program_id(1)

    @pl.when(k == 0)
    def _():
        o_ref[...] = jnp.zeros_like(o_ref)

    part = c_ref[1] * jnp.dot(adj_ref[...], x_ref[...],
                              preferred_element_type=jnp.float32)
    for i, d_ref in enumerate((d1_ref, d2_ref, d3_ref)):
        part += jnp.dot(d_ref[0], z_ref[i], preferred_element_type=jnp.float32)
    o_ref[...] += part

    @pl.when(k == pl.num_programs(1) - 1)
    def _():
        s = c_ref[3] * o_ref[...] + c_ref[2] * h0_ref[...]
        o_ref[...] = (c_ref[4] * jnp.dot(s, w_ref[...],
                                         preferred_element_type=jnp.float32)
                      + c_ref[5] * s)


def kernel(input, adj, d_list, h0, weight, lamda, alpha, l, gamma):
    x = input
    rv2 = jax.random.uniform(jax.random.key(42), (_LEV * _R * _N, 1),
                             dtype=jnp.float32)[_N:]
    theta = jnp.log(lamda / l + 1)
    g = jnp.asarray(gamma, jnp.float32)
    a = jnp.asarray(alpha, jnp.float32)
    t = jnp.asarray(theta, jnp.float32)
    c = jnp.stack([g, 1 - g, a, 1 - a, t, 1 - t]).astype(jnp.float32)

    nrb = _N // _BM1
    z = pl.pallas_call(
        _z_kernel,
        grid=(_NS // _BM1,),
        in_specs=[
            pl.BlockSpec(memory_space=pltpu.SMEM),
            pl.BlockSpec((_BM1, 1), lambda m: (m, 0)),
            pl.BlockSpec((1, _BM1, _N), lambda m: (1 + m // nrb, m % nrb, 0)),
            pl.BlockSpec((_N, _F), lambda m: (0, 0)),
        ],
        out_specs=pl.BlockSpec((_BM1, _F), lambda m: (m, 0)),
        out_shape=jax.ShapeDtypeStruct((_NS, _F), jnp.float32),
    )(c, rv2, d_list, x)

    z3 = z.reshape(_NOP, _N, _F)

    out = pl.pallas_call(
        _out_kernel,
        grid=(_N // _BM2, _N // _BK2),
        in_specs=[
            pl.BlockSpec(memory_space=pltpu.SMEM),
            pl.BlockSpec((_BM2, _BK2), lambda m, k: (m, k)),
            pl.BlockSpec((1, _BM2, _BK2), lambda m, k: (1, m, k)),
            pl.BlockSpec((1, _BM2, _BK2), lambda m, k: (2, m, k)),
            pl.BlockSpec((1, _BM2, _BK2), lambda m, k: (3, m, k)),
            pl.BlockSpec((_NOP, _BK2, _F), lambda m, k: (0, k, 0)),
            pl.BlockSpec((_BK2, _F), lambda m, k: (k, 0)),
            pl.BlockSpec((_BM2, _F), lambda m, k: (m, 0)),
            pl.BlockSpec((_F, _F), lambda m, k: (0, 0)),
        ],
        out_specs=pl.BlockSpec((_BM2, _F), lambda m, k: (m, 0)),
        out_shape=jax.ShapeDtypeStruct((_N, _F), jnp.float32),
    )(c, adj, d_list, d_list, d_list, z3, x, h0, weight)
    return out


# speedup vs baseline: 1.7418x; 1.3092x over previous
import jax
import jax.numpy as jnp
from jax.experimental import pallas as pl
from jax.experimental.pallas import tpu as pltpu

_N = 2048
_F = 256
_LEV = 2
_R = 2
_NOP = _LEV * _R - 1
_NS = _NOP * _N

_BM1 = 512
_BM2 = 256
_BK2 = 512


def _z_kernel(c_ref, rv_ref, d_ref, x_ref, z_ref):
    z_ref[...] = (c_ref[0] * rv_ref[...]) * jnp.dot(
        d_ref[0], x_ref[...], preferred_element_type=jnp.float32)


def _out_kernel(c_ref, adj_ref, d1_ref, d2_ref, d3_ref, z_ref, x_ref, h0_ref,
                w_ref, o_ref):
    acc = c_ref[1] * jnp.dot(adj_ref[...], x_ref[...],
                             preferred_element_type=jnp.float32)
    for i, d_ref in enumerate((d1_ref, d2_ref, d3_ref)):
        acc += jnp.dot(d_ref[0], z_ref[i], preferred_element_type=jnp.float32)
    s = c_ref[3] * acc + c_ref[2] * h0_ref[...]
    o_ref[...] = (c_ref[4] * jnp.dot(s, w_ref[...],
                                     preferred_element_type=jnp.float32)
                  + c_ref[5] * s)


def kernel(input, adj, d_list, h0, weight, lamda, alpha, l, gamma):
    x = input
    rv2 = jax.random.uniform(jax.random.key(42), (_LEV * _R * _N, 1),
                             dtype=jnp.float32)[_N:]
    theta = jnp.log(lamda / l + 1)
    g = jnp.asarray(gamma, jnp.float32)
    a = jnp.asarray(alpha, jnp.float32)
    t = jnp.asarray(theta, jnp.float32)
    c = jnp.stack([g, 1 - g, a, 1 - a, t, 1 - t]).astype(jnp.float32)

    nrb = _N // _BM1
    z = pl.pallas_call(
        _z_kernel,
        grid=(_NS // _BM1,),
        in_specs=[
            pl.BlockSpec(memory_space=pltpu.SMEM),
            pl.BlockSpec((_BM1, 1), lambda m: (m, 0)),
            pl.BlockSpec((1, _BM1, _N), lambda m: (1 + m // nrb, m % nrb, 0)),
            pl.BlockSpec((_N, _F), lambda m: (0, 0)),
        ],
        out_specs=pl.BlockSpec((_BM1, _F), lambda m: (m, 0)),
        out_shape=jax.ShapeDtypeStruct((_NS, _F), jnp.float32),
    )(c, rv2, d_list, x)

    z3 = z.reshape(_NOP, _N, _F)

    out = pl.pallas_call(
        _out_kernel,
        grid=(_N // _BM2,),
        in_specs=[
            pl.BlockSpec(memory_space=pltpu.SMEM),
            pl.BlockSpec((_BM2, _N), lambda m: (m, 0)),
            pl.BlockSpec((1, _BM2, _N), lambda m: (1, m, 0)),
            pl.BlockSpec((1, _BM2, _N), lambda m: (2, m, 0)),
            pl.BlockSpec((1, _BM2, _N), lambda m: (3, m, 0)),
            pl.BlockSpec((_NOP, _N, _F), lambda m: (0, 0, 0)),
            pl.BlockSpec((_N, _F), lambda m: (0, 0)),
            pl.BlockSpec((_BM2, _F), lambda m: (m, 0)),
            pl.BlockSpec((_F, _F), lambda m: (0, 0)),
        ],
        out_specs=pl.BlockSpec((_BM2, _F), lambda m: (m, 0)),
        out_shape=jax.ShapeDtypeStruct((_N, _F), jnp.float32),
    )(c, adj, d_list, d_list, d_list, z3, x, h0, weight)
    return out


# speedup vs baseline: 1.7778x; 1.0207x over previous
import jax
import jax.numpy as jnp
from jax.experimental import pallas as pl
from jax.experimental.pallas import tpu as pltpu

_N = 2048
_F = 256
_LEV = 2
_R = 2
_NOP = _LEV * _R - 1
_NS = _NOP * _N

_BM1 = 256
_BM2 = 128
_P1 = _NS // _BM1
_P2 = _N // _BM2
_NRB = _N // _BM1


def _fused_kernel(c_ref, rv_ref, d_ref, adj_ref, x_ref, h0_ref, w_ref, o_ref,
                  dv_ref, z_ref):
    p = pl.program_id(0)

    @pl.when(p < _P1)
    def _():
        blk = d_ref[0]
        row = p * _BM1
        dv_ref[pl.ds(row, _BM1), :] = blk
        z_ref[pl.ds(row, _BM1), :] = (c_ref[0] * rv_ref[...]) * jnp.dot(
            blk, x_ref[...], preferred_element_type=jnp.float32)

    @pl.when(p >= _P1)
    def _():
        m = p - _P1
        acc = c_ref[1] * jnp.dot(adj_ref[...], x_ref[...],
                                 preferred_element_type=jnp.float32)
        for i in range(_NOP):
            dblk = dv_ref[pl.ds(i * _N + m * _BM2, _BM2), :]
            acc += jnp.dot(dblk, z_ref[pl.ds(i * _N, _N), :],
                           preferred_element_type=jnp.float32)
        s = c_ref[3] * acc + c_ref[2] * h0_ref[...]
        o_ref[...] = (c_ref[4] * jnp.dot(s, w_ref[...],
                                         preferred_element_type=jnp.float32)
                      + c_ref[5] * s)


def kernel(input, adj, d_list, h0, weight, lamda, alpha, l, gamma):
    x = input
    rv2 = jax.random.uniform(jax.random.key(42), (_LEV * _R * _N, 1),
                             dtype=jnp.float32)[_N:]
    theta = jnp.log(lamda / l + 1)
    g = jnp.asarray(gamma, jnp.float32)
    a = jnp.asarray(alpha, jnp.float32)
    t = jnp.asarray(theta, jnp.float32)
    c = jnp.stack([g, 1 - g, a, 1 - a, t, 1 - t]).astype(jnp.float32)

    out = pl.pallas_call(
        _fused_kernel,
        grid=(_P1 + _P2,),
        in_specs=[
            pl.BlockSpec(memory_space=pltpu.SMEM),
            pl.BlockSpec((_BM1, 1),
                         lambda p: (jnp.minimum(p, _P1 - 1), 0)),
            pl.BlockSpec((1, _BM1, _N),
                         lambda p: (1 + jnp.minimum(p, _P1 - 1) // _NRB,
                                    jnp.minimum(p, _P1 - 1) % _NRB, 0)),
            pl.BlockSpec((_BM2, _N),
                         lambda p: (jnp.maximum(p - _P1, 0), 0)),
            pl.BlockSpec((_N, _F), lambda p: (0, 0)),
            pl.BlockSpec((_BM2, _F),
                         lambda p: (jnp.maximum(p - _P1, 0), 0)),
            pl.BlockSpec((_F, _F), lambda p: (0, 0)),
        ],
        out_specs=pl.BlockSpec((_BM2, _F),
                               lambda p: (jnp.maximum(p - _P1, 0), 0)),
        out_shape=jax.ShapeDtypeStruct((_N, _F), jnp.float32),
        compiler_params=pltpu.CompilerParams(vmem_limit_bytes=67_000_000),
        scratch_shapes=[
            pltpu.VMEM((_NS, _N), jnp.float32),
            pltpu.VMEM((_NS, _F), jnp.float32),
        ],
    )(c, rv2, d_list, adj, x, h0, weight)
    return out


# speedup vs baseline: 2.1404x; 1.2039x over previous
import jax
import jax.numpy as jnp
from jax.experimental import pallas as pl
from jax.experimental.pallas import tpu as pltpu

_N = 2048
_F = 256
_LEV = 2
_R = 2
_NOP = _LEV * _R - 1
_NS = _NOP * _N

_BM1 = 512
_BM2 = 256
_P1 = _NS // _BM1
_P2 = _N // _BM2
_NRB = _N // _BM1


def _fused_kernel(c_ref, rv_ref, d_ref, adj_ref, x_ref, h0_ref, w_ref, o_ref,
                  dv_ref, z_ref, xbf_ref):
    p = pl.program_id(0)

    @pl.when(p == 0)
    def _():
        xbf_ref[...] = x_ref[...].astype(jnp.bfloat16)

    @pl.when(p < _P1)
    def _():
        blk = d_ref[0].astype(jnp.bfloat16)
        row = p * _BM1
        dv_ref[pl.ds(row, _BM1), :] = blk
        zblk = (c_ref[0] * rv_ref[...]) * jnp.dot(
            blk, xbf_ref[...], preferred_element_type=jnp.float32)
        z_ref[pl.ds(row, _BM1), :] = zblk.astype(jnp.bfloat16)

    @pl.when(p >= _P1)
    def _():
        m = p - _P1
        acc = c_ref[1] * jnp.dot(adj_ref[...].astype(jnp.bfloat16),
                                 xbf_ref[...],
                                 preferred_element_type=jnp.float32)
        for i in range(_NOP):
            dblk = dv_ref[pl.ds(i * _N + m * _BM2, _BM2), :]
            acc += jnp.dot(dblk, z_ref[pl.ds(i * _N, _N), :],
                           preferred_element_type=jnp.float32)
        s = c_ref[3] * acc + c_ref[2] * h0_ref[...]
        o_ref[...] = (c_ref[4] * jnp.dot(s, w_ref[...],
                                         preferred_element_type=jnp.float32)
                      + c_ref[5] * s)


def kernel(input, adj, d_list, h0, weight, lamda, alpha, l, gamma):
    x = input
    rv2 = jax.random.uniform(jax.random.key(42), (_LEV * _R * _N, 1),
                             dtype=jnp.float32)[_N:]
    theta = jnp.log(lamda / l + 1)
    g = jnp.asarray(gamma, jnp.float32)
    a = jnp.asarray(alpha, jnp.float32)
    t = jnp.asarray(theta, jnp.float32)
    c = jnp.stack([g, 1 - g, a, 1 - a, t, 1 - t]).astype(jnp.float32)

    out = pl.pallas_call(
        _fused_kernel,
        grid=(_P1 + _P2,),
        in_specs=[
            pl.BlockSpec(memory_space=pltpu.SMEM),
            pl.BlockSpec((_BM1, 1),
                         lambda p: (jnp.minimum(p, _P1 - 1), 0)),
            pl.BlockSpec((1, _BM1, _N),
                         lambda p: (1 + jnp.minimum(p, _P1 - 1) // _NRB,
                                    jnp.minimum(p, _P1 - 1) % _NRB, 0)),
            pl.BlockSpec((_BM2, _N),
                         lambda p: (jnp.maximum(p - _P1, 0), 0)),
            pl.BlockSpec((_N, _F), lambda p: (0, 0)),
            pl.BlockSpec((_BM2, _F),
                         lambda p: (jnp.maximum(p - _P1, 0), 0)),
            pl.BlockSpec((_F, _F), lambda p: (0, 0)),
        ],
        out_specs=pl.BlockSpec((_BM2, _F),
                               lambda p: (jnp.maximum(p - _P1, 0), 0)),
        out_shape=jax.ShapeDtypeStruct((_N, _F), jnp.float32),
        compiler_params=pltpu.CompilerParams(vmem_limit_bytes=67_000_000),
        scratch_shapes=[
            pltpu.VMEM((_NS, _N), jnp.bfloat16),
            pltpu.VMEM((_NS, _F), jnp.bfloat16),
            pltpu.VMEM((_N, _F), jnp.bfloat16),
        ],
    )(c, rv2, d_list, adj, x, h0, weight)
    return out
